# initial kernel scaffold (unmeasured)
import jax
import jax.numpy as jnp
from jax import lax
from jax.experimental import pallas as pl
from jax.experimental.pallas import tpu as pltpu

N_DEV = 4
N_GLOBAL = 4096
EPS = 1e-5


def kernel(x, gamma):
    m, n = x.shape
    assert m % 128 == 0
    sub = m // 128

    def body(x_ref, g_ref, out_ref, comm_ref, send_sems, recv_sems):
        my = lax.axis_index("i")

        bar = pltpu.get_barrier_semaphore()
        for j in range(1, N_DEV):
            pl.semaphore_signal(
                bar, inc=1,
                device_id=((my + j) % N_DEV,),
                device_id_type=pl.DeviceIdType.MESH,
            )
        pl.semaphore_wait(bar, N_DEV - 1)

        xf = x_ref[...]
        partial = jnp.sum(xf * xf, axis=1)
        comm_ref[0] = partial.reshape(sub, 128)

        rdmas = []
        for j in range(1, N_DEV):
            rdma = pltpu.make_async_remote_copy(
                src_ref=comm_ref.at[0],
                dst_ref=comm_ref.at[j],
                send_sem=send_sems.at[j - 1],
                recv_sem=recv_sems.at[j - 1],
                device_id=((my + j) % N_DEV,),
                device_id_type=pl.DeviceIdType.MESH,
            )
            rdma.start()
            rdmas.append(rdma)

        scaled = xf * g_ref[...]

        for rdma in rdmas:
            rdma.wait()

        total = (
            comm_ref[0] + comm_ref[1] + comm_ref[2] + comm_ref[3]
        )
        rinv = lax.rsqrt(total / N_GLOBAL + EPS)
        rinv_col = rinv.reshape(m, 1)
        out_ref[...] = (scaled * rinv_col).astype(out_ref.dtype)

    return pl.pallas_call(
        body,
        out_shape=jax.ShapeDtypeStruct((m, n), jnp.bfloat16),
        in_specs=[
            pl.BlockSpec(memory_space=pltpu.VMEM),
            pl.BlockSpec(memory_space=pltpu.VMEM),
        ],
        out_specs=pl.BlockSpec(memory_space=pltpu.VMEM),
        scratch_shapes=[
            pltpu.VMEM((N_DEV, sub, 128), jnp.float32),
            pltpu.SemaphoreType.DMA((N_DEV - 1,)),
            pltpu.SemaphoreType.DMA((N_DEV - 1,)),
        ],
        compiler_params=pltpu.CompilerParams(collective_id=0),
    )(x, gamma.reshape(1, n))


# baseline (device time: 13488 ns/iter reference)
import jax
import jax.numpy as jnp
from jax import lax
from jax.experimental import pallas as pl
from jax.experimental.pallas import tpu as pltpu

N_DEV = 4
N_GLOBAL = 4096
EPS = 1e-5
BLK = 128


def kernel(x, gamma):
    m, n = x.shape
    nblk = m // BLK

    def body(x_ref, g_ref, out_ref, t_ref, comm_ref, send_sems, recv_sems):
        my = lax.axis_index("i")

        bar = pltpu.get_barrier_semaphore()
        for j in range(1, N_DEV):
            pl.semaphore_signal(
                bar, inc=1,
                device_id=((my + j) % N_DEV,),
                device_id_type=pl.DeviceIdType.MESH,
            )
        pl.semaphore_wait(bar, N_DEV - 1)

        g = g_ref[...]

        for s in range(nblk):
            blk = x_ref[pl.ds(s * BLK, BLK), :]
            comm_ref[0, :, s : s + 1] = jnp.sum(
                blk * blk, axis=1, keepdims=True
            )

        rdmas = []
        for j in range(1, N_DEV):
            rdma = pltpu.make_async_remote_copy(
                src_ref=comm_ref.at[0],
                dst_ref=comm_ref.at[j],
                send_sem=send_sems.at[j - 1],
                recv_sem=recv_sems.at[j - 1],
                device_id=((my + j) % N_DEV,),
                device_id_type=pl.DeviceIdType.MESH,
            )
            rdma.start()
            rdmas.append(rdma)

        for s in range(nblk):
            t_ref[pl.ds(s * BLK, BLK), :] = x_ref[pl.ds(s * BLK, BLK), :] * g

        for rdma in rdmas:
            rdma.wait()

        total = (
            comm_ref[0] + comm_ref[1] + comm_ref[2] + comm_ref[3]
        )
        rinv = lax.rsqrt(total / N_GLOBAL + EPS)
        for s in range(nblk):
            out_ref[pl.ds(s * BLK, BLK), :] = (
                t_ref[pl.ds(s * BLK, BLK), :] * rinv[:, s : s + 1]
            ).astype(out_ref.dtype)

    return pl.pallas_call(
        body,
        out_shape=jax.ShapeDtypeStruct((m, n), jnp.bfloat16),
        in_specs=[
            pl.BlockSpec(memory_space=pltpu.VMEM),
            pl.BlockSpec(memory_space=pltpu.VMEM),
        ],
        out_specs=pl.BlockSpec(memory_space=pltpu.VMEM),
        scratch_shapes=[
            pltpu.VMEM((m, n), jnp.float32),
            pltpu.VMEM((N_DEV, BLK, nblk), jnp.float32),
            pltpu.SemaphoreType.DMA((N_DEV - 1,)),
            pltpu.SemaphoreType.DMA((N_DEV - 1,)),
        ],
        compiler_params=pltpu.CompilerParams(collective_id=0),
    )(x, gamma.reshape(1, n))
